# baseline (speedup 1.0000x reference)
import jax
import jax.numpy as jnp
from jax import lax
from jax.experimental import pallas as pl
from jax.experimental.pallas import tpu as pltpu
from jax.experimental.pallas import tpu_sc as plsc

BATCH = 16384
EMBED_DIM = 32
NUM_WORKERS = 32
PER_WORKER = BATCH // NUM_WORKERS
LANES = 16
GROUPS = PER_WORKER // LANES
WAVE = 4
WPG = LANES // WAVE
NWAVES = GROUPS * WPG
NSLOTGRP = 3
TCOL = 128


def _mf_kernel(user_ids, item_ids, user_table_t, item_table_t, user_bias,
               item_bias, out_hbm, uid_v, iid_v, uslab_v, islab_v,
               ubias_v, ibias_v, ucols_v, icols_v, out_v, sems, bsem):
    wid = lax.axis_index("s") * 2 + lax.axis_index("c")
    base = wid * PER_WORKER

    pltpu.sync_copy(user_ids.at[pl.ds(base, PER_WORKER)], uid_v)
    pltpu.sync_copy(item_ids.at[pl.ds(base, PER_WORKER)], iid_v)

    lane_iota = lax.iota(jnp.int32, LANES)

    def fire_wave(w, uvec, ivec, klane):
        slot = lax.rem(w, NSLOTGRP) * WAVE
        for s in range(WAVE):
            j = klane * WAVE + s
            ucol = pl.multiple_of((uvec[j] // TCOL) * TCOL, TCOL)
            icol = pl.multiple_of((ivec[j] // TCOL) * TCOL, TCOL)
            pltpu.async_copy(
                user_table_t.at[:, pl.ds(ucol, TCOL)], uslab_v.at[slot + s],
                sems.at[lax.rem(w, NSLOTGRP)])
            pltpu.async_copy(
                item_table_t.at[:, pl.ds(icol, TCOL)], islab_v.at[slot + s],
                sems.at[lax.rem(w, NSLOTGRP)])

    def wait_wave(w):
        slot = lax.rem(w, NSLOTGRP) * WAVE
        for s in range(WAVE):
            pltpu.make_async_copy(
                user_table_t.at[:, pl.ds(0, TCOL)], uslab_v.at[slot + s],
                sems.at[lax.rem(w, NSLOTGRP)]).wait()
            pltpu.make_async_copy(
                item_table_t.at[:, pl.ds(0, TCOL)], islab_v.at[slot + s],
                sems.at[lax.rem(w, NSLOTGRP)]).wait()

    def extract_wave(w, k, uu, iu):
        slot = lax.rem(w, NSLOTGRP) * WAVE
        for s in range(WAVE):
            j = k * WAVE + s
            svec = jnp.full((LANES,), slot + s, jnp.int32)
            uj = jnp.full((LANES,), uu[j], jnp.int32)
            ij = jnp.full((LANES,), iu[j], jnp.int32)
            lo = lane_iota
            hi = lane_iota + LANES
            ucols_v[j, pl.ds(0, LANES)] = plsc.load_gather(
                uslab_v, [svec, lo, uj])
            ucols_v[j, pl.ds(LANES, LANES)] = plsc.load_gather(
                uslab_v, [svec, hi, uj])
            icols_v[j, pl.ds(0, LANES)] = plsc.load_gather(
                islab_v, [svec, lo, ij])
            icols_v[j, pl.ds(LANES, LANES)] = plsc.load_gather(
                islab_v, [svec, hi, ij])

    fire_wave(jnp.int32(0), uid_v[pl.ds(0, LANES)], iid_v[pl.ds(0, LANES)], 0)

    def group_body(g, _):
        b0 = g * LANES
        uvec = uid_v[pl.ds(b0, LANES)]
        ivec = iid_v[pl.ds(b0, LANES)]
        uvec_n = uid_v[pl.ds(lax.min(b0 + LANES, PER_WORKER - LANES), LANES)]
        ivec_n = iid_v[pl.ds(lax.min(b0 + LANES, PER_WORKER - LANES), LANES)]

        bias_copies = []
        for j in range(LANES):
            ucol = pl.multiple_of((uvec[j] // TCOL) * TCOL, TCOL)
            icol = pl.multiple_of((ivec[j] // TCOL) * TCOL, TCOL)
            bias_copies.append(pltpu.async_copy(
                user_bias.at[pl.ds(ucol, TCOL)], ubias_v.at[j], bsem))
            bias_copies.append(pltpu.async_copy(
                item_bias.at[pl.ds(icol, TCOL)], ibias_v.at[j], bsem))

        uu = uvec - (uvec // TCOL) * TCOL
        iu = ivec - (ivec // TCOL) * TCOL

        for k in range(WPG):
            w = g * WPG + k
            wait_wave(w)
            if k < WPG - 1:
                fire_wave(w + 1, uvec, ivec, k + 1)
            else:
                @pl.when(g + 1 < GROUPS)
                def _():
                    fire_wave(w + 1, uvec_n, ivec_n, 0)
            extract_wave(w, k, uu, iu)

        for cp in bias_copies:
            cp.wait()

        acc = (plsc.load_gather(ubias_v, [lane_iota, uu])
               + plsc.load_gather(ibias_v, [lane_iota, iu]))
        for d in range(EMBED_DIM):
            dvec = jnp.full((LANES,), d, jnp.int32)
            u = plsc.load_gather(ucols_v, [lane_iota, dvec])
            it = plsc.load_gather(icols_v, [lane_iota, dvec])
            acc = acc + u * it
        out_v[pl.ds(b0, LANES)] = acc
        return 0

    lax.fori_loop(0, GROUPS, group_body, 0)

    pltpu.sync_copy(out_v, out_hbm.at[pl.ds(base, PER_WORKER)])


@jax.jit
def _mf(user_ids, item_ids, user_table_t, item_table_t, user_bias, item_bias):
    mesh = plsc.VectorSubcoreMesh(core_axis_name="c", subcore_axis_name="s")
    kfn = pl.kernel(
        _mf_kernel,
        mesh=mesh,
        compiler_params=pltpu.CompilerParams(needs_layout_passes=False),
        out_type=jax.ShapeDtypeStruct((BATCH,), jnp.float32),
        scratch_types=[
            pltpu.VMEM((PER_WORKER,), jnp.int32),
            pltpu.VMEM((PER_WORKER,), jnp.int32),
            pltpu.VMEM((NSLOTGRP * WAVE, EMBED_DIM, TCOL), jnp.float32),
            pltpu.VMEM((NSLOTGRP * WAVE, EMBED_DIM, TCOL), jnp.float32),
            pltpu.VMEM((LANES, TCOL), jnp.float32),
            pltpu.VMEM((LANES, TCOL), jnp.float32),
            pltpu.VMEM((LANES, 2 * LANES), jnp.float32),
            pltpu.VMEM((LANES, 2 * LANES), jnp.float32),
            pltpu.VMEM((PER_WORKER,), jnp.float32),
            pltpu.SemaphoreType.DMA((NSLOTGRP,)),
            pltpu.SemaphoreType.DMA,
        ],
    )
    return kfn(user_ids, item_ids, user_table_t, item_table_t, user_bias,
               item_bias)


def kernel(user_ids, item_ids, user_table, item_table, user_bias, item_bias):
    return _mf(user_ids, item_ids, user_table.T, item_table.T,
               user_bias.reshape(-1), item_bias.reshape(-1))


# speedup vs baseline: 1.0185x; 1.0185x over previous
import jax
import jax.numpy as jnp
from jax import lax
from jax.experimental import pallas as pl
from jax.experimental.pallas import tpu as pltpu
from jax.experimental.pallas import tpu_sc as plsc

BATCH = 16384
EMBED_DIM = 32
NUM_WORKERS = 32
PER_WORKER = BATCH // NUM_WORKERS
LANES = 16
GROUPS = PER_WORKER // LANES
WAVE = 8
TCOL = 128


def _mf_kernel(user_ids, item_ids, user_table_t, item_table_t, user_bias,
               item_bias, out_hbm, uid_v, iid_v, uslab_v, islab_v,
               ubias_v, ibias_v, ucols_v, icols_v, out_v, sem, bsem):
    wid = lax.axis_index("s") * 2 + lax.axis_index("c")
    base = wid * PER_WORKER

    pltpu.sync_copy(user_ids.at[pl.ds(base, PER_WORKER)], uid_v)
    pltpu.sync_copy(item_ids.at[pl.ds(base, PER_WORKER)], iid_v)

    lane_iota = lax.iota(jnp.int32, LANES)

    def group_body(g, _):
        b0 = g * LANES
        uvec = uid_v[pl.ds(b0, LANES)]
        ivec = iid_v[pl.ds(b0, LANES)]

        bias_copies = []
        for j in range(LANES):
            ucol = pl.multiple_of((uvec[j] // TCOL) * TCOL, TCOL)
            icol = pl.multiple_of((ivec[j] // TCOL) * TCOL, TCOL)
            bias_copies.append(pltpu.async_copy(
                user_bias.at[pl.ds(ucol, TCOL)], ubias_v.at[j], bsem))
            bias_copies.append(pltpu.async_copy(
                item_bias.at[pl.ds(icol, TCOL)], ibias_v.at[j], bsem))

        def fire_wave(w):
            copies = []
            for s in range(WAVE):
                j = w * WAVE + s
                ucol = pl.multiple_of((uvec[j] // TCOL) * TCOL, TCOL)
                icol = pl.multiple_of((ivec[j] // TCOL) * TCOL, TCOL)
                for gg in range(EMBED_DIM // 8):
                    fsl = pl.ds(gg * 8, 8)
                    copies.append(pltpu.async_copy(
                        user_table_t.at[fsl, pl.ds(ucol, TCOL)],
                        uslab_v.at[s, fsl], sem))
                    copies.append(pltpu.async_copy(
                        item_table_t.at[fsl, pl.ds(icol, TCOL)],
                        islab_v.at[s, fsl], sem))
            return copies

        def extract_wave(w, uu, iu):
            for s in range(WAVE):
                j = w * WAVE + s
                svec = jnp.full((LANES,), s, jnp.int32)
                uj = jnp.full((LANES,), uu[j], jnp.int32)
                ij = jnp.full((LANES,), iu[j], jnp.int32)
                lo = lane_iota
                hi = lane_iota + LANES
                ucols_v[j, pl.ds(0, LANES)] = plsc.load_gather(
                    uslab_v, [svec, lo, uj])
                ucols_v[j, pl.ds(LANES, LANES)] = plsc.load_gather(
                    uslab_v, [svec, hi, uj])
                icols_v[j, pl.ds(0, LANES)] = plsc.load_gather(
                    islab_v, [svec, lo, ij])
                icols_v[j, pl.ds(LANES, LANES)] = plsc.load_gather(
                    islab_v, [svec, hi, ij])

        uu = uvec - (uvec // TCOL) * TCOL
        iu = ivec - (ivec // TCOL) * TCOL

        w0 = fire_wave(0)
        for cp in w0:
            cp.wait()
        extract_wave(0, uu, iu)
        w1 = fire_wave(1)
        for cp in w1:
            cp.wait()
        extract_wave(1, uu, iu)
        for cp in bias_copies:
            cp.wait()

        acc = (plsc.load_gather(ubias_v, [lane_iota, uu])
               + plsc.load_gather(ibias_v, [lane_iota, iu]))
        for d in range(EMBED_DIM):
            dvec = jnp.full((LANES,), d, jnp.int32)
            u = plsc.load_gather(ucols_v, [lane_iota, dvec])
            it = plsc.load_gather(icols_v, [lane_iota, dvec])
            acc = acc + u * it
        out_v[pl.ds(b0, LANES)] = acc
        return 0

    lax.fori_loop(0, GROUPS, group_body, 0)

    pltpu.sync_copy(out_v, out_hbm.at[pl.ds(base, PER_WORKER)])


@jax.jit
def _mf(user_ids, item_ids, user_table_t, item_table_t, user_bias, item_bias):
    mesh = plsc.VectorSubcoreMesh(core_axis_name="c", subcore_axis_name="s")
    kfn = pl.kernel(
        _mf_kernel,
        mesh=mesh,
        compiler_params=pltpu.CompilerParams(needs_layout_passes=False),
        out_type=jax.ShapeDtypeStruct((BATCH,), jnp.float32),
        scratch_types=[
            pltpu.VMEM((PER_WORKER,), jnp.int32),
            pltpu.VMEM((PER_WORKER,), jnp.int32),
            pltpu.VMEM((WAVE, EMBED_DIM, TCOL), jnp.float32),
            pltpu.VMEM((WAVE, EMBED_DIM, TCOL), jnp.float32),
            pltpu.VMEM((LANES, TCOL), jnp.float32),
            pltpu.VMEM((LANES, TCOL), jnp.float32),
            pltpu.VMEM((LANES, 2 * LANES), jnp.float32),
            pltpu.VMEM((LANES, 2 * LANES), jnp.float32),
            pltpu.VMEM((PER_WORKER,), jnp.float32),
            pltpu.SemaphoreType.DMA,
            pltpu.SemaphoreType.DMA,
        ],
    )
    return kfn(user_ids, item_ids, user_table_t, item_table_t, user_bias,
               item_bias)


def kernel(user_ids, item_ids, user_table, item_table, user_bias, item_bias):
    return _mf(user_ids, item_ids, user_table.T, item_table.T,
               user_bias.reshape(-1), item_bias.reshape(-1))
